# Initial kernel scaffold; baseline (speedup 1.0000x reference)
import jax
import jax.numpy as jnp
from jax.experimental import pallas as pl


def kernel(input, embed_weight):
    raise NotImplementedError("write your pallas kernel here")


# speedup vs baseline: 1.5774x; 1.5774x over previous
import functools

import jax
import jax.numpy as jnp
from jax import lax
from jax.experimental import pallas as pl
from jax.experimental.pallas import tpu as pltpu
from jax.experimental.pallas import tpu_sc as plsc

EMB = 32
ROWS, COLS = 16384, 26
B = ROWS * COLS
CHUNK = 128
NGROUPS = B // CHUNK
NC, NS = 2, 16
NW = NC * NS
G_PER_W = NGROUPS // NW
NBUF = 8

_mesh = plsc.VectorSubcoreMesh(
    core_axis_name="c", subcore_axis_name="s", num_cores=NC, num_subcores=NS
)


@functools.partial(
    pl.kernel,
    out_type=jax.ShapeDtypeStruct((B, EMB), jnp.float32),
    mesh=_mesh,
    scratch_types=[
        pltpu.VMEM((G_PER_W, CHUNK), jnp.int32),
        pltpu.VMEM((NBUF, CHUNK, EMB), jnp.float32),
        pltpu.SemaphoreType.DMA,
    ],
    compiler_params=pltpu.CompilerParams(use_tc_tiling_on_sc=False),
)
def _embedding_gather(idx_hbm, table_hbm, out_hbm, idx_v, bufs, gsem):
    wid = lax.axis_index("s") * NC + lax.axis_index("c")
    g0 = wid * G_PER_W
    pltpu.sync_copy(idx_hbm.at[pl.ds(g0, G_PER_W)], idx_v)

    for b in range(NBUF):
        pltpu.async_copy(table_hbm.at[idx_v.at[b]], bufs.at[b], gsem)

    @pl.loop(0, G_PER_W, step=NBUF)
    def _outer(g):
        for b in range(NBUF):
            cur = g + b
            pltpu.make_async_copy(
                table_hbm.at[idx_v.at[cur]], bufs.at[b], gsem
            ).wait()
            pltpu.sync_copy(
                bufs.at[b], out_hbm.at[pl.ds((g0 + cur) * CHUNK, CHUNK)]
            )
            nxt = cur + NBUF

            @pl.when(nxt < G_PER_W)
            def _refill():
                pltpu.async_copy(table_hbm.at[idx_v.at[nxt]], bufs.at[b], gsem)


def kernel(input, embed_weight):
    idx = input.reshape(NGROUPS, CHUNK)
    out = _embedding_gather(idx, embed_weight)
    return out.reshape(ROWS, COLS, EMB)
